# Initial kernel scaffold; baseline (speedup 1.0000x reference)
import jax
import jax.numpy as jnp
from jax.experimental import pallas as pl


def kernel(x, edge_index, batch, W0, b0, convW, convb, gW1, gb1, gW2, gb2, mW0, mb0, mW1, mb1, mW2, mb2, mW3, mb3):
    raise NotImplementedError("write your pallas kernel here")


# speedup vs baseline: 9.8969x; 9.8969x over previous
import functools

import jax
import jax.numpy as jnp
from jax import lax
from jax.experimental import pallas as pl
from jax.experimental.pallas import tpu as pltpu
from jax.experimental.pallas import tpu_sc as plsc

N = 10000
E = 320000
F_IN = 128
D = 64
G = 16

NC = 2
NS = 16
NW = NC * NS
SUB = 632
NP = NS * SUB
CW = 128
KCH = 79
EPW = KCH * CW
EP = NW * EPW


def _sc_degrees_body(srcb, dstb, ones16, z16, out, idx_s, idx_d, ones_v,
                     dsrc_sh, ddst_sh):
    c = lax.axis_index("c")
    s = lax.axis_index("s")
    w = c * NS + s
    pltpu.sync_copy(z16, dsrc_sh.at[pl.ds(s * SUB, SUB)])
    pltpu.sync_copy(z16, ddst_sh.at[pl.ds(s * SUB, SUB)])
    pltpu.sync_copy(ones16, ones_v)
    pltpu.sync_copy(srcb.at[w], idx_s)
    pltpu.sync_copy(dstb.at[w], idx_d)
    plsc.subcore_barrier()

    def body(j, carry):
        pltpu.sync_copy(ones_v, dsrc_sh.at[idx_s.at[j]], add=True)
        pltpu.sync_copy(ones_v, ddst_sh.at[idx_d.at[j]], add=True)
        return carry

    lax.fori_loop(0, KCH, body, 0)
    plsc.subcore_barrier()
    pltpu.sync_copy(dsrc_sh.at[pl.ds(s * SUB, SUB)], out.at[c, 0, pl.ds(s * SUB, SUB)])
    pltpu.sync_copy(ddst_sh.at[pl.ds(s * SUB, SUB)], out.at[c, 1, pl.ds(s * SUB, SUB)])


def _sc_scatter_body(tbl, srcb, dstb, z64, out, idx_s, idx_d, rows, agg_sh, gsem):
    c = lax.axis_index("c")
    s = lax.axis_index("s")
    w = c * NS + s
    pltpu.sync_copy(z64, agg_sh.at[pl.ds(s * SUB, SUB)])
    pltpu.sync_copy(srcb.at[w], idx_s)
    pltpu.sync_copy(dstb.at[w], idx_d)
    plsc.subcore_barrier()

    def body(j, carry):
        pltpu.async_copy(tbl.at[idx_s.at[j]], rows, gsem).wait()
        pltpu.sync_copy(rows, agg_sh.at[idx_d.at[j]], add=True)
        return carry

    lax.fori_loop(0, KCH, body, 0)
    plsc.subcore_barrier()
    pltpu.sync_copy(agg_sh.at[pl.ds(s * SUB, SUB)], out.at[c, pl.ds(s * SUB, SUB)])


@functools.lru_cache(maxsize=None)
def _sc_kernels():
    mesh = plsc.VectorSubcoreMesh(
        core_axis_name="c", subcore_axis_name="s", num_cores=NC, num_subcores=NS
    )
    params = pltpu.CompilerParams(use_tc_tiling_on_sc=False)
    sc_degrees = pl.kernel(
        _sc_degrees_body,
        out_type=jax.ShapeDtypeStruct((NC, 2, NP, 16), jnp.float32),
        mesh=mesh,
        compiler_params=params,
        scratch_types=[
            pltpu.VMEM((KCH, CW), jnp.int32),
            pltpu.VMEM((KCH, CW), jnp.int32),
            pltpu.VMEM((CW, 16), jnp.float32),
            pltpu.VMEM_SHARED((NP, 16), jnp.float32),
            pltpu.VMEM_SHARED((NP, 16), jnp.float32),
        ],
    )
    sc_scatter = pl.kernel(
        _sc_scatter_body,
        out_type=jax.ShapeDtypeStruct((NC, NP, D), jnp.float32),
        mesh=mesh,
        compiler_params=params,
        scratch_types=[
            pltpu.VMEM((KCH, CW), jnp.int32),
            pltpu.VMEM((KCH, CW), jnp.int32),
            pltpu.VMEM((CW, D), jnp.float32),
            pltpu.VMEM_SHARED((NP, D), jnp.float32),
            pltpu.SemaphoreType.DMA,
        ],
    )
    return sc_degrees, sc_scatter


def _elu(v):
    return jnp.where(v > 0.0, v, jnp.exp(jnp.minimum(v, 0.0)) - 1.0)


def _tc_pre_body(degp_ref, x_ref, w0_ref, tbl_ref, ds_ref, dd_ref):
    deg = degp_ref[0] + degp_ref[1]
    ds = lax.rsqrt(jnp.maximum(deg[0, :, 0], 1.0))
    dd = lax.rsqrt(jnp.maximum(deg[1, :, 0], 1.0))
    ds_ref[...] = ds[:, None]
    dd_ref[...] = dd[:, None]
    h = jnp.dot(x_ref[...], w0_ref[...], preferred_element_type=jnp.float32)
    tbl_ref[...] = h * ds[:, None]


_tc_pre = pl.pallas_call(
    _tc_pre_body,
    out_shape=(
        jax.ShapeDtypeStruct((NP, D), jnp.float32),
        jax.ShapeDtypeStruct((NP, 1), jnp.float32),
        jax.ShapeDtypeStruct((NP, 1), jnp.float32),
    ),
)


def _tc_post_body(first, aggp_ref, dd_ref, b_ref, ds_ref, wn_ref, jk_in_ref,
                  jk_ref, tbl_ref):
    agg = aggp_ref[0] + aggp_ref[1]
    h = _elu(agg * dd_ref[...] + b_ref[...])
    if first:
        jk = h
    else:
        jk = jnp.maximum(jk_in_ref[...], h)
    jk_ref[...] = jk
    hw = jnp.dot(h, wn_ref[...], preferred_element_type=jnp.float32)
    tbl_ref[...] = hw * ds_ref[...]


_tc_post_first = pl.pallas_call(
    lambda aggp, dd, b, ds, wn, jk, tbl: _tc_post_body(
        True, aggp, dd, b, ds, wn, None, jk, tbl),
    out_shape=(
        jax.ShapeDtypeStruct((NP, D), jnp.float32),
        jax.ShapeDtypeStruct((NP, D), jnp.float32),
    ),
)

_tc_post_mid = pl.pallas_call(
    lambda aggp, dd, b, ds, wn, jk_in, jk, tbl: _tc_post_body(
        False, aggp, dd, b, ds, wn, jk_in, jk, tbl),
    out_shape=(
        jax.ShapeDtypeStruct((NP, D), jnp.float32),
        jax.ShapeDtypeStruct((NP, D), jnp.float32),
    ),
)


def _tc_final_body(aggp_ref, dd_ref, b_ref, jk_in_ref, batch_ref,
                   gw1_ref, gb1_ref, gw2_ref, gb2_ref,
                   mw0_ref, mb0_ref, mw1_ref, mb1_ref,
                   mw2_ref, mb2_ref, mw3_ref, mb3_ref, out_ref):
    agg = aggp_ref[0] + aggp_ref[1]
    h = _elu(agg * dd_ref[...] + b_ref[...])
    jk = jnp.maximum(jk_in_ref[...], h)

    g1 = jnp.maximum(
        jnp.dot(jk, gw1_ref[...], preferred_element_type=jnp.float32)
        + gb1_ref[...], 0.0)
    gate = (jnp.dot(g1, gw2_ref[...], preferred_element_type=jnp.float32)
            + gb2_ref[...])

    batch = batch_ref[...]
    gids = lax.broadcasted_iota(jnp.int32, (NP, G), 1)
    ohb = batch == gids
    oh = ohb.astype(jnp.float32)

    gmax = jnp.max(jnp.where(ohb, gate, -1e30), axis=0)
    gmaxb = jnp.dot(oh, gmax[:, None], preferred_element_type=jnp.float32)
    valid = (batch < G).astype(jnp.float32)
    ex = jnp.exp(gate - gmaxb) * valid
    gsum = lax.dot_general(oh, ex, (((0,), (0,)), ((), ())),
                           preferred_element_type=jnp.float32)
    gsumb = jnp.dot(oh, gsum, preferred_element_type=jnp.float32)
    att = ex / (gsumb + 1e-16)
    pooled = lax.dot_general(oh, jk * att, (((0,), (0,)), ((), ())),
                             preferred_element_type=jnp.float32)

    m = _elu(jnp.dot(pooled, mw0_ref[...], preferred_element_type=jnp.float32)
             + mb0_ref[...])
    m = _elu(jnp.dot(m, mw1_ref[...], preferred_element_type=jnp.float32)
             + mb1_ref[...])
    m = _elu(jnp.dot(m, mw2_ref[...], preferred_element_type=jnp.float32)
             + mb2_ref[...])
    out_ref[...] = (jnp.dot(m, mw3_ref[...], preferred_element_type=jnp.float32)
                    + mb3_ref[...])


_tc_final = pl.pallas_call(
    _tc_final_body,
    out_shape=jax.ShapeDtypeStruct((G, 1), jnp.float32),
)


def kernel(x, edge_index, batch, W0, b0, convW, convb, gW1, gb1, gW2, gb2,
           mW0, mb0, mW1, mb1, mW2, mb2, mW3, mb3):
    src = edge_index[0]
    dst = edge_index[1]
    pad_e = jnp.full((EP - E,), N, jnp.int32)
    src_b = jnp.concatenate([src, pad_e]).reshape(NW, KCH, CW)
    dst_b = jnp.concatenate([dst, pad_e]).reshape(NW, KCH, CW)
    x_p = jnp.zeros((NP, F_IN), jnp.float32).at[:N].set(x)
    batch_p = jnp.full((NP, 1), G, jnp.int32).at[:N, 0].set(batch)

    ones16 = jnp.ones((CW, 16), jnp.float32)
    z16 = jnp.zeros((SUB, 16), jnp.float32)
    z64 = jnp.zeros((SUB, D), jnp.float32)

    sc_degrees, sc_scatter = _sc_kernels()
    degp = sc_degrees(src_b, dst_b, ones16, z16)
    tbl, ds, dd = _tc_pre(degp, x_p, W0)

    jk = None
    out = None
    for i in range(6):
        aggp = sc_scatter(tbl, src_b, dst_b, z64)
        if i == 0:
            jk, tbl = _tc_post_first(aggp, dd, b0.reshape(1, D), ds, convW[0])
        elif i < 5:
            jk, tbl = _tc_post_mid(aggp, dd, convb[i - 1].reshape(1, D), ds,
                                   convW[i], jk)
        else:
            out = _tc_final(aggp, dd, convb[4].reshape(1, D), jk, batch_p,
                            gW1, gb1.reshape(1, D), gW2, gb2.reshape(1, 1),
                            mW0, mb0.reshape(1, 32), mW1, mb1.reshape(1, 16),
                            mW2, mb2.reshape(1, 8), mW3, mb3.reshape(1, 1))
    return out
